# baseline (speedup 1.0000x reference)
import functools

import jax
import jax.numpy as jnp
from jax import lax
from jax.experimental import pallas as pl
from jax.experimental.pallas import tpu as pltpu
from jax.experimental.pallas import tpu_sc as plsc

HEADS = 4
H = 1024
V = 32000
B = 128
THRESH = 0.09

VT = 3200
NVT = V // VT
CW = 128
CPT = VT // CW
NCH = V // CW
NCHP = 256
NROWS = HEADS * B
NWORK = 32
RPW = NROWS // NWORK
NEG = -3.0e38


def _tc_body(x_ref, wres_ref, bres_ref, wlm_ref,
             logits_ref, cmax_ref, csum_ref, h_ref):
    vt = pl.program_id(1)

    @pl.when(vt == 0)
    def _():
        x = x_ref[...]
        pre = lax.dot_general(x, wres_ref[0], (((1,), (1,)), ((), ())),
                              preferred_element_type=jnp.float32)
        pre = pre + bres_ref[0]
        h_ref[...] = x + pre * jax.nn.sigmoid(pre)

    logits = lax.dot_general(h_ref[...], wlm_ref[0], (((1,), (1,)), ((), ())),
                             preferred_element_type=jnp.float32)
    logits_ref[0] = logits

    mx_cols = []
    sm_cols = []
    for c in range(CPT):
        seg = logits[:, c * CW:(c + 1) * CW]
        mx = jnp.max(seg, axis=1, keepdims=True)
        sm = jnp.sum(jnp.exp(seg - mx), axis=1, keepdims=True)
        mx_cols.append(mx)
        sm_cols.append(sm)
    cmax_ref[0, 0] = jnp.concatenate(mx_cols, axis=1)
    csum_ref[0, 0] = jnp.concatenate(sm_cols, axis=1)


def _tc_phase(x, W_res, b_res, W_lm):
    return pl.pallas_call(
        _tc_body,
        grid=(HEADS, NVT),
        in_specs=[
            pl.BlockSpec((B, H), lambda h, v: (0, 0)),
            pl.BlockSpec((1, H, H), lambda h, v: (h, 0, 0)),
            pl.BlockSpec((1, 1, H), lambda h, v: (h, 0, 0)),
            pl.BlockSpec((1, VT, H), lambda h, v: (h, v, 0)),
        ],
        out_specs=[
            pl.BlockSpec((1, B, VT), lambda h, v: (h, 0, v)),
            pl.BlockSpec((1, 1, B, CPT), lambda h, v: (h, v, 0, 0)),
            pl.BlockSpec((1, 1, B, CPT), lambda h, v: (h, v, 0, 0)),
        ],
        out_shape=[
            jax.ShapeDtypeStruct((HEADS, B, V), jnp.float32),
            jax.ShapeDtypeStruct((HEADS, NVT, B, CPT), jnp.float32),
            jax.ShapeDtypeStruct((HEADS, NVT, B, CPT), jnp.float32),
        ],
        scratch_shapes=[pltpu.VMEM((B, H), jnp.float32)],
    )(x, W_res, b_res.reshape(HEADS, 1, H), W_lm)


def _sc_phase(logits_rows, cmax, csum, nrows=NROWS):
    rpw = nrows // NWORK
    mesh = plsc.VectorSubcoreMesh(core_axis_name="c", subcore_axis_name="s")

    def _lane_max(v):
        m = v[0]
        for i in range(1, 16):
            m = jnp.maximum(m, v[i])
        return m

    def _lane_sum(v):
        s = v[0]
        for i in range(1, 16):
            s = s + v[i]
        return s

    def _popcount(mask):
        return plsc.all_reduce_population_count(mask)[0]

    @functools.partial(
        pl.kernel,
        mesh=mesh,
        compiler_params=pltpu.CompilerParams(needs_layout_passes=False),
        out_type=[
            jax.ShapeDtypeStruct((nrows, 16), jnp.float32),
            jax.ShapeDtypeStruct((nrows, 16), jnp.int32),
        ],
        scratch_types=[
            pltpu.VMEM((rpw, NCHP), jnp.float32),
            pltpu.VMEM((rpw, NCHP), jnp.float32),
            pltpu.VMEM((64,), jnp.int32),
            pltpu.VMEM((16, CW), jnp.float32),
            pltpu.VMEM((48,), jnp.float32),
            pltpu.VMEM((48,), jnp.int32),
            pltpu.VMEM((16,), jnp.float32),
            pltpu.VMEM((16,), jnp.int32),
            pltpu.SMEM((4,), jnp.int32),
            pltpu.SemaphoreType.DMA,
        ],
    )
    def k(logits_hbm, cmax_hbm, csum_hbm, vals_hbm, idx_hbm,
          cmax_v, csum_v, sel_v, rows_v, cval_v, cidx_v, ov_v, oi_v,
          cnt_s, sem):
        wid = lax.axis_index("s") * 2 + lax.axis_index("c")
        base = wid * rpw
        pltpu.sync_copy(cmax_hbm.at[pl.ds(base, rpw)], cmax_v)
        pltpu.sync_copy(csum_hbm.at[pl.ds(base, rpw)], csum_v)

        def row_body(r, carry):
            gr = base + r
            mvec = jnp.full((16,), NEG, jnp.float32)
            for j in range(NCHP // 16):
                mvec = jnp.maximum(mvec, cmax_v[r, pl.ds(j * 16, 16)])
            m = _lane_max(mvec)
            zvec = jnp.zeros((16,), jnp.float32)
            for j in range(NCHP // 16):
                cm = cmax_v[r, pl.ds(j * 16, 16)]
                cs = csum_v[r, pl.ds(j * 16, 16)]
                zvec = zvec + jnp.exp(cm - m) * cs
            z = _lane_sum(zvec)
            t = THRESH * z
            for j in range(4):
                sel_v[pl.ds(j * 16, 16)] = jnp.full((16,), gr * NCH, jnp.int32)
            cnt_s[0] = 0
            for j in range(NCHP // 16):
                cm = cmax_v[r, pl.ds(j * 16, 16)]
                e = jnp.exp(cm - m)
                sel = (e >= t) | (cm == m)
                cid = gr * NCH + j * 16 + lax.iota(jnp.int32, 16)
                cnt = cnt_s[0]
                pref = plsc.cumsum(sel.astype(jnp.int32))
                dest = jnp.minimum(jnp.where(sel, cnt + pref - 1, 63), 63)
                plsc.store_scatter(sel_v, [dest], cid)
                cnt_s[0] = cnt + _popcount(sel)
            nsel = jnp.minimum(cnt_s[0], 16)
            pltpu.async_copy(logits_hbm.at[sel_v.at[pl.ds(0, 16)]],
                             rows_v, sem).wait()
            for j in range(3):
                cval_v[pl.ds(j * 16, 16)] = jnp.zeros((16,), jnp.float32)
                cidx_v[pl.ds(j * 16, 16)] = jnp.full((16,), -1, jnp.int32)
            cnt_s[1] = 0

            def slot_body(s, carry2):
                valid = s < nsel
                cid_vec = sel_v[pl.ds(s, 16)]
                vbase = (cid_vec[0] - gr * NCH) * CW
                for j in range(CW // 16):
                    v = rows_v[s, pl.ds(j * 16, 16)]
                    e = jnp.exp(v - m)
                    cand = ((e >= t) | (v == m)) & valid
                    vid = vbase + j * 16 + lax.iota(jnp.int32, 16)
                    cc = cnt_s[1]
                    pref = plsc.cumsum(cand.astype(jnp.int32))
                    dest = jnp.minimum(jnp.where(cand, cc + pref - 1, 47), 47)
                    plsc.store_scatter(cval_v, [dest], e)
                    plsc.store_scatter(cidx_v, [dest], vid)
                    cnt_s[1] = cc + _popcount(cand)
                return carry2

            lax.fori_loop(0, 16, slot_body, 0)
            ev = cval_v[pl.ds(0, 16)]
            iv = cidx_v[pl.ds(0, 16)]
            sv, si = plsc.sort_key_val(ev, iv, descending=True)
            ov_v[...] = sv / z
            oi_v[...] = si
            pltpu.sync_copy(ov_v, vals_hbm.at[gr])
            pltpu.sync_copy(oi_v, idx_hbm.at[gr])
            return carry

        lax.fori_loop(0, rpw, row_body, 0)

    return k(logits_rows, cmax, csum)


def kernel(hidden_states, W_res, b_res, W_lm, k):
    vals_h = []
    idx_h = []
    for h in range(HEADS):
        lg, cm3, cs3 = _tc_phase_one(hidden_states, W_res[h],
                                     b_res[h].reshape(1, 1, H), W_lm[h])
        lr = lg.reshape(B * NCH, CW)
        cm = cm3.transpose(1, 0, 2).reshape(B, NCH)
        cs = cs3.transpose(1, 0, 2).reshape(B, NCH)
        cm = jnp.concatenate(
            [cm, jnp.full((B, NCHP - NCH), NEG, jnp.float32)], axis=1)
        cs = jnp.concatenate(
            [cs, jnp.zeros((B, NCHP - NCH), jnp.float32)], axis=1)
        v_h, i_h = _sc_phase(lr, cm, cs, nrows=B)
        vals_h.append(v_h)
        idx_h.append(i_h)
    vals3 = jnp.stack(vals_h)[:, :, :10]
    idx3 = jnp.stack(idx_h)[:, :, :10]
    pos = jnp.arange(10)[None, None, :]
    keep = ((vals3 >= THRESH) | (pos == 0)) & (pos < k)
    return jnp.where(keep, vals3, 0.0), jnp.where(keep, idx3, -1)


def _kernel_fused(hidden_states, W_res, b_res, W_lm, k):
    logits, cmax4, csum4 = _tc_phase(hidden_states, W_res, b_res, W_lm)
    logits_rows = logits.reshape(NROWS * NCH, CW)
    cmax = cmax4.transpose(0, 2, 1, 3).reshape(NROWS, NCH)
    csum = csum4.transpose(0, 2, 1, 3).reshape(NROWS, NCH)
    cmax = jnp.concatenate(
        [cmax, jnp.full((NROWS, NCHP - NCH), NEG, jnp.float32)], axis=1)
    csum = jnp.concatenate(
        [csum, jnp.zeros((NROWS, NCHP - NCH), jnp.float32)], axis=1)
    vals, idx = _sc_phase(logits_rows, cmax, csum)
    vals3 = vals.reshape(HEADS, B, 16)[:, :, :10]
    idx3 = idx.reshape(HEADS, B, 16)[:, :, :10]
    pos = jnp.arange(10)[None, None, :]
    keep = ((vals3 >= THRESH) | (pos == 0)) & (pos < k)
    return jnp.where(keep, vals3, 0.0), jnp.where(keep, idx3, -1)


def _tc_phase_one(x, W_res_h, b_res_h, W_lm_h):
    def body(x_ref, wres_ref, bres_ref, wlm_ref,
             logits_ref, cmax_ref, csum_ref, h_ref):
        vt = pl.program_id(0)

        @pl.when(vt == 0)
        def _():
            xv = x_ref[...]
            pre = lax.dot_general(xv, wres_ref[...], (((1,), (1,)), ((), ())),
                                  preferred_element_type=jnp.float32)
            pre = pre + bres_ref[0]
            h_ref[...] = xv + pre * jax.nn.sigmoid(pre)

        logits = lax.dot_general(h_ref[...], wlm_ref[...],
                                 (((1,), (1,)), ((), ())),
                                 preferred_element_type=jnp.float32)
        logits_ref[...] = logits
        mx_cols = []
        sm_cols = []
        for c in range(CPT):
            seg = logits[:, c * CW:(c + 1) * CW]
            mx = jnp.max(seg, axis=1, keepdims=True)
            sm = jnp.sum(jnp.exp(seg - mx), axis=1, keepdims=True)
            mx_cols.append(mx)
            sm_cols.append(sm)
        cmax_ref[0] = jnp.concatenate(mx_cols, axis=1)
        csum_ref[0] = jnp.concatenate(sm_cols, axis=1)

    return pl.pallas_call(
        body,
        grid=(NVT,),
        in_specs=[
            pl.BlockSpec((B, H), lambda v: (0, 0)),
            pl.BlockSpec((H, H), lambda v: (0, 0)),
            pl.BlockSpec((1, 1, H), lambda v: (0, 0, 0)),
            pl.BlockSpec((VT, H), lambda v: (v, 0)),
        ],
        out_specs=[
            pl.BlockSpec((B, VT), lambda v: (0, v)),
            pl.BlockSpec((1, B, CPT), lambda v: (v, 0, 0)),
            pl.BlockSpec((1, B, CPT), lambda v: (v, 0, 0)),
        ],
        out_shape=[
            jax.ShapeDtypeStruct((B, V), jnp.float32),
            jax.ShapeDtypeStruct((NVT, B, CPT), jnp.float32),
            jax.ShapeDtypeStruct((NVT, B, CPT), jnp.float32),
        ],
        scratch_shapes=[pltpu.VMEM((B, H), jnp.float32)],
    )(x, W_res_h, b_res_h, W_lm_h)


# speedup vs baseline: 2.0586x; 2.0586x over previous
import functools

import jax
import jax.numpy as jnp
from jax import lax
from jax.experimental import pallas as pl
from jax.experimental.pallas import tpu as pltpu
from jax.experimental.pallas import tpu_sc as plsc

HEADS = 4
H = 1024
V = 32000
B = 128
THRESH = 0.09

VT = 3200
NVT = V // VT
CW = 128
CPT = VT // CW
NCH = V // CW
NCHP = 256
NROWS = HEADS * B
NWORK = 32
RPW = NROWS // NWORK
NEG = -3.0e38


def _tc_body(x_ref, wres_ref, bres_ref, wlm_ref,
             logits_ref, cmax_ref, csum_ref, h_ref):
    vt = pl.program_id(1)

    @pl.when(vt == 0)
    def _():
        x = x_ref[...]
        pre = lax.dot_general(x, wres_ref[0], (((1,), (1,)), ((), ())),
                              preferred_element_type=jnp.float32)
        pre = pre + bres_ref[0]
        h_ref[...] = x + pre * jax.nn.sigmoid(pre)

    logits = lax.dot_general(h_ref[...], wlm_ref[0], (((1,), (1,)), ((), ())),
                             preferred_element_type=jnp.float32)
    logits_ref[0] = logits

    mx_cols = []
    sm_cols = []
    for c in range(CPT):
        seg = logits[:, c * CW:(c + 1) * CW]
        mx = jnp.max(seg, axis=1, keepdims=True)
        sm = jnp.sum(jnp.exp(seg - mx), axis=1, keepdims=True)
        mx_cols.append(mx)
        sm_cols.append(sm)
    cmax_ref[0, 0] = jnp.concatenate(mx_cols, axis=1)
    csum_ref[0, 0] = jnp.concatenate(sm_cols, axis=1)


def _tc_phase(x, W_res, b_res, W_lm):
    return pl.pallas_call(
        _tc_body,
        grid=(HEADS, NVT),
        in_specs=[
            pl.BlockSpec((B, H), lambda h, v: (0, 0)),
            pl.BlockSpec((1, H, H), lambda h, v: (h, 0, 0)),
            pl.BlockSpec((1, 1, H), lambda h, v: (h, 0, 0)),
            pl.BlockSpec((1, VT, H), lambda h, v: (h, v, 0)),
        ],
        out_specs=[
            pl.BlockSpec((1, B, VT), lambda h, v: (h, 0, v)),
            pl.BlockSpec((1, 1, B, CPT), lambda h, v: (h, v, 0, 0)),
            pl.BlockSpec((1, 1, B, CPT), lambda h, v: (h, v, 0, 0)),
        ],
        out_shape=[
            jax.ShapeDtypeStruct((HEADS, B, V), jnp.float32),
            jax.ShapeDtypeStruct((HEADS, NVT, B, CPT), jnp.float32),
            jax.ShapeDtypeStruct((HEADS, NVT, B, CPT), jnp.float32),
        ],
        scratch_shapes=[pltpu.VMEM((B, H), jnp.float32)],
    )(x, W_res, b_res.reshape(HEADS, 1, H), W_lm)


def _sc_phase(logits_rows, cmax, csum):
    mesh = plsc.VectorSubcoreMesh(core_axis_name="c", subcore_axis_name="s")

    def _lane_max(v):
        m = v[0]
        for i in range(1, 16):
            m = jnp.maximum(m, v[i])
        return m

    def _lane_sum(v):
        s = v[0]
        for i in range(1, 16):
            s = s + v[i]
        return s

    @functools.partial(
        pl.kernel,
        mesh=mesh,
        compiler_params=pltpu.CompilerParams(needs_layout_passes=False),
        out_type=[
            jax.ShapeDtypeStruct((NROWS, 16), jnp.float32),
            jax.ShapeDtypeStruct((NROWS, 16), jnp.int32),
        ],
        scratch_types=[
            pltpu.VMEM((RPW, NCHP), jnp.float32),
            pltpu.VMEM((RPW, NCHP), jnp.float32),
            pltpu.VMEM((RPW * 64,), jnp.int32),
            pltpu.VMEM((RPW, 16, CW), jnp.float32),
            pltpu.VMEM((48,), jnp.float32),
            pltpu.VMEM((48,), jnp.int32),
            pltpu.VMEM((RPW, 16), jnp.float32),
            pltpu.VMEM((RPW, 16), jnp.int32),
            pltpu.SMEM((RPW,), jnp.float32),
            pltpu.SMEM((RPW,), jnp.float32),
            pltpu.SMEM((RPW,), jnp.int32),
            pltpu.SMEM((4,), jnp.int32),
            pltpu.SemaphoreType.DMA,
        ],
    )
    def k(logits_hbm, cmax_hbm, csum_hbm, vals_hbm, idx_hbm,
          cmax_v, csum_v, sel_v, rows_v, cval_v, cidx_v, ov_v, oi_v,
          m_s, z_s, n_s, cnt_s, sem):
        wid = lax.axis_index("s") * 2 + lax.axis_index("c")
        base = wid * RPW
        pltpu.sync_copy(cmax_hbm.at[pl.ds(base, RPW)], cmax_v)
        pltpu.sync_copy(csum_hbm.at[pl.ds(base, RPW)], csum_v)

        def stat_body(r, carry):
            gr = base + r
            mvec = jnp.full((16,), NEG, jnp.float32)
            for j in range(NCHP // 16):
                mvec = jnp.maximum(mvec, cmax_v[r, pl.ds(j * 16, 16)])
            m = _lane_max(mvec)
            zvec = jnp.zeros((16,), jnp.float32)
            for j in range(NCHP // 16):
                cm = cmax_v[r, pl.ds(j * 16, 16)]
                cs = csum_v[r, pl.ds(j * 16, 16)]
                zvec = zvec + jnp.exp(cm - m) * cs
            z = _lane_sum(zvec)
            t = THRESH * z
            m_s[r] = m
            z_s[r] = z
            rb = r * 64
            for j in range(4):
                sel_v[pl.ds(rb + j * 16, 16)] = jnp.full((16,), gr * NCH,
                                                         jnp.int32)
            cnt_s[0] = 0
            for j in range(NCHP // 16):
                cm = cmax_v[r, pl.ds(j * 16, 16)]
                e = jnp.exp(cm - m)
                sel = (e >= t) | (cm == m)
                cid = gr * NCH + j * 16 + lax.iota(jnp.int32, 16)
                cnt = cnt_s[0]
                pref = plsc.cumsum(sel.astype(jnp.int32))
                dest = jnp.minimum(jnp.where(sel, rb + cnt + pref - 1, rb + 63),
                                   rb + 63)
                plsc.store_scatter(sel_v, [dest], cid)
                cnt_s[0] = cnt + pref[15]
            n_s[r] = jnp.minimum(cnt_s[0], 16)
            pltpu.make_async_copy(logits_hbm.at[sel_v.at[pl.ds(rb, 16)]],
                                  rows_v.at[r], sem).start()
            return carry

        lax.fori_loop(0, RPW, stat_body, 0)

        def drain_body(r, carry):
            rb = r * 64
            pltpu.make_async_copy(logits_hbm.at[sel_v.at[pl.ds(rb, 16)]],
                                  rows_v.at[r], sem).wait()
            return carry

        lax.fori_loop(0, RPW, drain_body, 0)

        def proc_body(r, carry):
            gr = base + r
            rb = r * 64
            m = m_s[r]
            z = z_s[r]
            t = THRESH * z
            nsel = n_s[r]
            for j in range(3):
                cval_v[pl.ds(j * 16, 16)] = jnp.zeros((16,), jnp.float32)
                cidx_v[pl.ds(j * 16, 16)] = jnp.full((16,), -1, jnp.int32)
            cnt_s[1] = 0

            def slot_body(s, carry2):
                valid = s < nsel
                cid_vec = sel_v[pl.ds(rb + s, 16)]
                vbase = (cid_vec[0] - gr * NCH) * CW
                for j in range(CW // 16):
                    v = rows_v[r, s, pl.ds(j * 16, 16)]
                    e = jnp.exp(v - m)
                    cand = ((e >= t) | (v == m)) & valid
                    vid = vbase + j * 16 + lax.iota(jnp.int32, 16)
                    cc = cnt_s[1]
                    pref = plsc.cumsum(cand.astype(jnp.int32))
                    dest = jnp.minimum(jnp.where(cand, cc + pref - 1, 47), 47)
                    plsc.store_scatter(cval_v, [dest], e)
                    plsc.store_scatter(cidx_v, [dest], vid)
                    cnt_s[1] = cc + pref[15]
                return carry2

            lax.fori_loop(0, 16, slot_body, 0)
            ev = cval_v[pl.ds(0, 16)]
            iv = cidx_v[pl.ds(0, 16)]
            sv, si = plsc.sort_key_val(ev, iv, descending=True)
            ov_v[r, pl.ds(0, 16)] = sv / z
            oi_v[r, pl.ds(0, 16)] = si
            return carry

        lax.fori_loop(0, RPW, proc_body, 0)
        pltpu.sync_copy(ov_v, vals_hbm.at[pl.ds(base, RPW)])
        pltpu.sync_copy(oi_v, idx_hbm.at[pl.ds(base, RPW)])

    return k(logits_rows, cmax, csum)


def kernel(hidden_states, W_res, b_res, W_lm, k):
    logits, cmax4, csum4 = _tc_phase(hidden_states, W_res, b_res, W_lm)
    logits_rows = logits.reshape(NROWS * NCH, CW)
    cmax = cmax4.transpose(0, 2, 1, 3).reshape(NROWS, NCH)
    csum = csum4.transpose(0, 2, 1, 3).reshape(NROWS, NCH)
    cmax = jnp.concatenate(
        [cmax, jnp.full((NROWS, NCHP - NCH), NEG, jnp.float32)], axis=1)
    csum = jnp.concatenate(
        [csum, jnp.zeros((NROWS, NCHP - NCH), jnp.float32)], axis=1)
    vals, idx = _sc_phase(logits_rows, cmax, csum)
    vals3 = vals.reshape(HEADS, B, 16)[:, :, :10]
    idx3 = idx.reshape(HEADS, B, 16)[:, :, :10]
    pos = jnp.arange(10)[None, None, :]
    keep = ((vals3 >= THRESH) | (pos == 0)) & (pos < k)
    return jnp.where(keep, vals3, 0.0), jnp.where(keep, idx3, -1)


# speedup vs baseline: 2.3866x; 1.1593x over previous
import functools

import jax
import jax.numpy as jnp
from jax import lax
from jax.experimental import pallas as pl
from jax.experimental.pallas import tpu as pltpu
from jax.experimental.pallas import tpu_sc as plsc

HEADS = 4
H = 1024
V = 32000
B = 128
THRESH = 0.09

VT = 3200
NVT = V // VT
CW = 128
CPT = VT // CW
NCH = V // CW
NCHP = 256
NROWS = HEADS * B
NWORK = 32
RPW = NROWS // NWORK
NEG = -3.0e38


def _tc_body(x_ref, wres_ref, bres_ref, wlm_ref,
             logits_ref, cmax_ref, csum_ref, h_ref):
    vt = pl.program_id(1)

    @pl.when(vt == 0)
    def _():
        x = x_ref[...]
        pre = lax.dot_general(x, wres_ref[0], (((1,), (1,)), ((), ())),
                              preferred_element_type=jnp.float32)
        pre = pre + bres_ref[0]
        h_ref[...] = x + pre * jax.nn.sigmoid(pre)

    logits = lax.dot_general(h_ref[...], wlm_ref[0], (((1,), (1,)), ((), ())),
                             preferred_element_type=jnp.float32)
    logits_ref[0] = logits

    mx_cols = []
    sm_cols = []
    for c in range(CPT):
        seg = logits[:, c * CW:(c + 1) * CW]
        mx = jnp.max(seg, axis=1, keepdims=True)
        sm = jnp.sum(jnp.exp(seg - mx), axis=1, keepdims=True)
        mx_cols.append(mx)
        sm_cols.append(sm)
    cmax_ref[0, 0] = jnp.concatenate(mx_cols, axis=1)
    csum_ref[0, 0] = jnp.concatenate(sm_cols, axis=1)


def _tc_phase(x, W_res, b_res, W_lm):
    return pl.pallas_call(
        _tc_body,
        grid=(HEADS, NVT),
        in_specs=[
            pl.BlockSpec((B, H), lambda h, v: (0, 0)),
            pl.BlockSpec((1, H, H), lambda h, v: (h, 0, 0)),
            pl.BlockSpec((1, 1, H), lambda h, v: (h, 0, 0)),
            pl.BlockSpec((1, VT, H), lambda h, v: (h, v, 0)),
        ],
        out_specs=[
            pl.BlockSpec((1, B, VT), lambda h, v: (h, 0, v)),
            pl.BlockSpec((1, 1, B, CPT), lambda h, v: (h, v, 0, 0)),
            pl.BlockSpec((1, 1, B, CPT), lambda h, v: (h, v, 0, 0)),
        ],
        out_shape=[
            jax.ShapeDtypeStruct((HEADS, B, V), jnp.float32),
            jax.ShapeDtypeStruct((HEADS, NVT, B, CPT), jnp.float32),
            jax.ShapeDtypeStruct((HEADS, NVT, B, CPT), jnp.float32),
        ],
        scratch_shapes=[pltpu.VMEM((B, H), jnp.float32)],
    )(x, W_res, b_res.reshape(HEADS, 1, H), W_lm)


def _sc_phase(logits_rows, cmax, csum):
    mesh = plsc.VectorSubcoreMesh(core_axis_name="c", subcore_axis_name="s")

    def _lane_max(v):
        m = v[0]
        for i in range(1, 16):
            m = jnp.maximum(m, v[i])
        return m

    def _lane_sum(v):
        s = v[0]
        for i in range(1, 16):
            s = s + v[i]
        return s

    @functools.partial(
        pl.kernel,
        mesh=mesh,
        compiler_params=pltpu.CompilerParams(needs_layout_passes=False),
        out_type=[
            jax.ShapeDtypeStruct((NROWS, 16), jnp.float32),
            jax.ShapeDtypeStruct((NROWS, 16), jnp.int32),
        ],
        scratch_types=[
            pltpu.VMEM((RPW, NCHP), jnp.float32),
            pltpu.VMEM((RPW, NCHP), jnp.float32),
            pltpu.VMEM((RPW * 64,), jnp.int32),
            pltpu.VMEM((RPW, 16, CW), jnp.float32),
            pltpu.VMEM((48,), jnp.float32),
            pltpu.VMEM((48,), jnp.int32),
            pltpu.VMEM((RPW, 16), jnp.float32),
            pltpu.VMEM((RPW, 16), jnp.int32),
            pltpu.SMEM((RPW,), jnp.float32),
            pltpu.SMEM((RPW,), jnp.float32),
            pltpu.SMEM((RPW,), jnp.int32),
            pltpu.SMEM((4,), jnp.int32),
            pltpu.SemaphoreType.DMA,
        ],
    )
    def k(logits_hbm, cmax_hbm, csum_hbm, vals_hbm, idx_hbm,
          cmax_v, csum_v, sel_v, rows_v, cval_v, cidx_v, ov_v, oi_v,
          m_s, z_s, n_s, cnt_s, sem):
        wid = lax.axis_index("s") * 2 + lax.axis_index("c")
        base = wid * RPW
        pltpu.sync_copy(cmax_hbm.at[pl.ds(base, RPW)], cmax_v)
        pltpu.sync_copy(csum_hbm.at[pl.ds(base, RPW)], csum_v)

        def stat_body(r, carry):
            gr = base + r
            mvec = jnp.full((16,), NEG, jnp.float32)
            for j in range(NCHP // 16):
                mvec = jnp.maximum(mvec, cmax_v[r, pl.ds(j * 16, 16)])
            m = _lane_max(mvec)
            zvec = jnp.zeros((16,), jnp.float32)
            for j in range(NCHP // 16):
                cm = cmax_v[r, pl.ds(j * 16, 16)]
                cs = csum_v[r, pl.ds(j * 16, 16)]
                zvec = zvec + jnp.exp(cm - m) * cs
            z = _lane_sum(zvec)
            t = THRESH * z
            m_s[r] = m
            z_s[r] = z
            rb = r * 64
            for j in range(4):
                sel_v[pl.ds(rb + j * 16, 16)] = jnp.full((16,), gr * NCH,
                                                         jnp.int32)
            cnt_s[0] = 0
            for j in range(NCHP // 16):
                cm = cmax_v[r, pl.ds(j * 16, 16)]
                e = jnp.exp(cm - m)
                sel = (e >= t) | (cm == m)
                cid = gr * NCH + j * 16 + lax.iota(jnp.int32, 16)
                cnt = cnt_s[0]
                pref = plsc.cumsum(sel.astype(jnp.int32))
                dest = jnp.minimum(jnp.where(sel, rb + cnt + pref - 1, rb + 63),
                                   rb + 63)
                plsc.store_scatter(sel_v, [dest], cid)
                cnt_s[0] = cnt + pref[15]
            n_s[r] = jnp.minimum(cnt_s[0], 16)
            pltpu.make_async_copy(logits_hbm.at[sel_v.at[pl.ds(rb, 16)]],
                                  rows_v.at[r], sem).start()
            return carry

        lax.fori_loop(0, RPW, stat_body, 0)

        def drain_body(r, carry):
            rb = r * 64
            pltpu.make_async_copy(logits_hbm.at[sel_v.at[pl.ds(rb, 16)]],
                                  rows_v.at[r], sem).wait()
            return carry

        lax.fori_loop(0, RPW, drain_body, 0)

        def proc_body(r, carry):
            gr = base + r
            rb = r * 64
            m = m_s[r]
            z = z_s[r]
            t = THRESH * z
            nsel = n_s[r]
            for j in range(3):
                cval_v[pl.ds(j * 16, 16)] = jnp.zeros((16,), jnp.float32)
                cidx_v[pl.ds(j * 16, 16)] = jnp.full((16,), -1, jnp.int32)
            cnt_s[1] = 0

            def slot_body(s, carry2):
                valid = s < nsel
                cid_vec = sel_v[pl.ds(rb + s, 16)]
                vbase = (cid_vec[0] - gr * NCH) * CW
                for j in range(CW // 16):
                    v = rows_v[r, s, pl.ds(j * 16, 16)]
                    e = jnp.exp(v - m)
                    cand = ((e >= t) | (v == m)) & valid
                    vid = vbase + j * 16 + lax.iota(jnp.int32, 16)
                    cc = cnt_s[1]
                    pref = plsc.cumsum(cand.astype(jnp.int32))
                    dest = jnp.minimum(jnp.where(cand, cc + pref - 1, 47), 47)
                    plsc.store_scatter(cval_v, [dest], e)
                    plsc.store_scatter(cidx_v, [dest], vid)
                    cnt_s[1] = cc + pref[15]
                return carry2

            lax.fori_loop(0, nsel, slot_body, 0)
            ev = cval_v[pl.ds(0, 16)]
            iv = cidx_v[pl.ds(0, 16)]
            sv, si = plsc.sort_key_val(ev, iv, descending=True)
            ov_v[r, pl.ds(0, 16)] = sv / z
            oi_v[r, pl.ds(0, 16)] = si
            return carry

        lax.fori_loop(0, RPW, proc_body, 0)
        pltpu.sync_copy(ov_v, vals_hbm.at[pl.ds(base, RPW)])
        pltpu.sync_copy(oi_v, idx_hbm.at[pl.ds(base, RPW)])

    return k(logits_rows, cmax, csum)


def kernel(hidden_states, W_res, b_res, W_lm, k):
    logits, cmax4, csum4 = _tc_phase(hidden_states, W_res, b_res, W_lm)
    logits_rows = logits.reshape(NROWS * NCH, CW)
    cmax = cmax4.transpose(0, 2, 1, 3).reshape(NROWS, NCH)
    csum = csum4.transpose(0, 2, 1, 3).reshape(NROWS, NCH)
    cmax = jnp.concatenate(
        [cmax, jnp.full((NROWS, NCHP - NCH), NEG, jnp.float32)], axis=1)
    csum = jnp.concatenate(
        [csum, jnp.zeros((NROWS, NCHP - NCH), jnp.float32)], axis=1)
    vals, idx = _sc_phase(logits_rows, cmax, csum)
    vals3 = vals.reshape(HEADS, B, 16)[:, :, :10]
    idx3 = idx.reshape(HEADS, B, 16)[:, :, :10]
    pos = jnp.arange(10)[None, None, :]
    keep = ((vals3 >= THRESH) | (pos == 0)) & (pos < k)
    return jnp.where(keep, vals3, 0.0), jnp.where(keep, idx3, -1)
